# baseline (speedup 1.0000x reference)
import jax
import jax.numpy as jnp
from jax import lax
from jax.experimental import pallas as pl
from jax.experimental.pallas import tpu as pltpu, tpu_sc as plsc

DEPTH = 1000
N = 51200
NW = 32
PER_W = N // NW
CH = 80
NCHUNK = PER_W // CH
GROUPS = CH // 16


def _sc_body(idx_hbm, zeros_hbm, out_hbm, idx_v, buf):
    wid = lax.axis_index("s") * 2 + lax.axis_index("c")
    base = wid * PER_W
    pltpu.sync_copy(idx_hbm.at[pl.ds(base, PER_W)], idx_v)
    pltpu.sync_copy(zeros_hbm, buf)
    iota = lax.iota(jnp.int32, 16)
    ones = jnp.full((16,), 1.0, jnp.float32)
    zeros = jnp.zeros((16,), jnp.float32)

    def chunk(c, carry):
        row0 = base + c * CH
        for j in range(GROUPS):
            rows = iota + j * 16
            cols = idx_v[pl.ds(c * CH + j * 16, 16)]
            plsc.store_scatter(buf, [rows, cols], ones)
        pltpu.sync_copy(buf, out_hbm.at[pl.ds(row0, CH)])
        for j in range(GROUPS):
            rows = iota + j * 16
            cols = idx_v[pl.ds(c * CH + j * 16, 16)]
            plsc.store_scatter(buf, [rows, cols], zeros)
        return carry

    lax.fori_loop(0, NCHUNK, chunk, 0)


def kernel(inputs):
    b, s = inputs.shape
    idx = inputs.astype(jnp.int32).reshape(N)
    zblock = jnp.zeros((CH, DEPTH), jnp.float32)
    mesh = plsc.VectorSubcoreMesh(core_axis_name="c", subcore_axis_name="s")
    k = pl.kernel(
        _sc_body,
        out_type=jax.ShapeDtypeStruct((N, DEPTH), jnp.float32),
        mesh=mesh,
        compiler_params=pltpu.CompilerParams(use_tc_tiling_on_sc=False, needs_layout_passes=False),
        scratch_types=[
            pltpu.VMEM((PER_W,), jnp.int32),
            pltpu.VMEM((CH, DEPTH), jnp.float32),
        ],
    )
    out = k(idx, zblock)
    return out.reshape(b, s, DEPTH)


# speedup vs baseline: 1.8533x; 1.8533x over previous
import jax
import jax.numpy as jnp
from jax.experimental import pallas as pl

DEPTH = 1000
BB = 8


def _onehot_body(idx_ref, out_ref):
    idx = idx_ref[...]
    iota = jax.lax.broadcasted_iota(jnp.int32, (BB, 50, DEPTH), 2)
    out_ref[...] = (idx[:, :, None] == iota).astype(jnp.float32)


def kernel(inputs):
    b, s = inputs.shape
    idx = inputs.astype(jnp.int32)
    out = pl.pallas_call(
        _onehot_body,
        grid=(b // BB,),
        in_specs=[pl.BlockSpec((BB, s), lambda i: (i, 0))],
        out_specs=pl.BlockSpec((BB, s, DEPTH), lambda i: (i, 0, 0)),
        out_shape=jax.ShapeDtypeStruct((b, s, DEPTH), jnp.float32),
    )(idx)
    return out
